# baseline (speedup 1.0000x reference)
import functools

import jax
import jax.numpy as jnp
from jax import lax
from jax.experimental import pallas as pl
from jax.experimental.pallas import tpu as pltpu
from jax.experimental.pallas import tpu_sc as plsc

_N = 10000
_E = 320000
_D = 128
_HID = 128
_NCLS = 64

_NSC = 2
_NTEC = 16
_NW = _NSC * _NTEC
_EPT = _E // _NW
_C = 80
_ITERS = _EPT // _C
_NB = 3

_R_STEP = 624
_R_LEN = 640

_mesh = plsc.VectorSubcoreMesh(core_axis_name="c", subcore_axis_name="s")


@functools.partial(
    pl.kernel,
    out_type=jax.ShapeDtypeStruct((_NSC * 2 * _N,), jnp.float32),
    mesh=_mesh,
    scratch_types=[
        pltpu.VMEM((_EPT,), jnp.int32),
        pltpu.VMEM((_EPT,), jnp.int32),
        pltpu.VMEM((_C,), jnp.float32),
        pltpu.VMEM((_R_LEN,), jnp.float32),
        pltpu.VMEM_SHARED((_N,), jnp.float32),
        pltpu.VMEM_SHARED((_N,), jnp.float32),
        pltpu.SemaphoreType.DMA,
        pltpu.SemaphoreType.DMA,
    ],
)
def _deg_kernel(src_h, dst_h, out_h,
                sidx, didx, ones_v, stage_v, dego_sp, degi_sp, sem0, sem1):
    cid = lax.axis_index("c")
    sid = lax.axis_index("s")
    wid = sid * _NSC + cid
    r0 = pl.multiple_of(sid * _R_STEP, 8)
    ebase = pl.multiple_of(wid * _EPT, 8)
    pltpu.sync_copy(src_h.at[pl.ds(ebase, _EPT)], sidx)
    pltpu.sync_copy(dst_h.at[pl.ds(ebase, _EPT)], didx)
    for j in range(_C // 16):
        ones_v[pl.ds(j * 16, 16)] = jnp.ones((16,), jnp.float32)
    for j in range(_R_LEN // 16):
        stage_v[pl.ds(j * 16, 16)] = jnp.zeros((16,), jnp.float32)
    pltpu.sync_copy(stage_v, dego_sp.at[pl.ds(r0, _R_LEN)])
    pltpu.sync_copy(stage_v, degi_sp.at[pl.ds(r0, _R_LEN)])
    plsc.subcore_barrier()

    def chunk(i):
        return pl.ds(pl.multiple_of(i * _C, 8), _C)

    def body(i, carry):
        pltpu.async_copy(ones_v, dego_sp.at[sidx.at[chunk(i)]], sem0, add=True)
        pltpu.async_copy(ones_v, degi_sp.at[didx.at[chunk(i)]], sem1, add=True)
        return carry

    lax.fori_loop(0, _ITERS, body, 0)

    def drain(i, carry):
        pltpu.make_async_copy(ones_v, dego_sp.at[sidx.at[chunk(0)]], sem0).wait()
        pltpu.make_async_copy(ones_v, degi_sp.at[didx.at[chunk(0)]], sem1).wait()
        return carry

    lax.fori_loop(0, _ITERS, drain, 0)
    plsc.subcore_barrier()
    o_base = pl.multiple_of(cid * (2 * _N) + r0, 8)
    pltpu.sync_copy(dego_sp.at[pl.ds(r0, _R_LEN)], stage_v)
    pltpu.sync_copy(stage_v, out_h.at[pl.ds(o_base, _R_LEN)])
    pltpu.sync_copy(degi_sp.at[pl.ds(r0, _R_LEN)], stage_v)
    pltpu.sync_copy(stage_v, out_h.at[pl.ds(o_base + _N, _R_LEN)])


def _make_agg(d):
    @functools.partial(
        pl.kernel,
        out_type=jax.ShapeDtypeStruct((_NSC, _N, d), jnp.float32),
        mesh=_mesh,
        scratch_types=[
            pltpu.VMEM((_EPT,), jnp.int32),
            pltpu.VMEM((_EPT,), jnp.int32),
            [pltpu.VMEM((_C, d), jnp.float32) for _ in range(_NB)],
            [pltpu.SemaphoreType.DMA for _ in range(_NB)],
            [pltpu.SemaphoreType.DMA for _ in range(_NB)],
            pltpu.VMEM_SHARED((_N, d), jnp.float32),
        ],
    )
    def _agg(p_h, src_h, dst_h, zeros_h, out_h, sidx, didx, rows, gsem, ssem,
             agg_sp):
        cid = lax.axis_index("c")
        sid = lax.axis_index("s")
        wid = sid * _NSC + cid
        r0 = pl.multiple_of(sid * _R_STEP, 8)
        ebase = pl.multiple_of(wid * _EPT, 8)
        pltpu.async_copy(src_h.at[pl.ds(ebase, _EPT)], sidx, gsem[0])
        pltpu.async_copy(dst_h.at[pl.ds(ebase, _EPT)], didx, gsem[1])
        pltpu.async_copy(zeros_h.at[pl.ds(r0, _R_LEN)],
                         agg_sp.at[pl.ds(r0, _R_LEN)], ssem[0])
        pltpu.make_async_copy(src_h.at[pl.ds(ebase, _EPT)], sidx, gsem[0]).wait()
        pltpu.make_async_copy(dst_h.at[pl.ds(ebase, _EPT)], didx, gsem[1]).wait()
        pltpu.make_async_copy(zeros_h.at[pl.ds(r0, _R_LEN)],
                              agg_sp.at[pl.ds(r0, _R_LEN)], ssem[0]).wait()
        plsc.subcore_barrier()

        def chunk(i):
            return pl.ds(pl.multiple_of(i * _C, 8), _C)

        def gstart(i, b):
            pltpu.async_copy(p_h.at[sidx.at[chunk(i)]], rows[b], gsem[b])

        def gwait(b):
            pltpu.make_async_copy(p_h.at[sidx.at[chunk(0)]], rows[b], gsem[b]).wait()

        def sstart(i, b):
            pltpu.async_copy(rows[b], agg_sp.at[didx.at[chunk(i)]], ssem[b],
                             add=True)

        def swait(b):
            pltpu.make_async_copy(rows[b], agg_sp.at[didx.at[chunk(0)]],
                                  ssem[b]).wait()

        gstart(0, 0)
        for i in range(2):
            gstart(i + 1, (i + 1) % _NB)
            gwait(i % _NB)
            sstart(i, i % _NB)

        def step(i, off):
            b = (2 + off) % _NB
            swait(off % _NB)
            gstart(i + 1, (3 + off) % _NB)
            gwait(b)
            sstart(i, b)

        def body(k, carry):
            for off in range(_NB):
                step(2 + _NB * k + off, off)
            return carry

        n_main = (_ITERS - 3 - 2) // _NB
        lax.fori_loop(0, n_main, body, 0)
        for j in range(_ITERS - 3 - _NB * n_main):
            step(2 + _NB * n_main + j, j)
        i_last = _ITERS - 1
        gwait(i_last % _NB)
        sstart(i_last, i_last % _NB)
        for b in range(_NB):
            swait(b)
        plsc.subcore_barrier()
        pltpu.sync_copy(agg_sp.at[pl.ds(r0, _R_LEN)],
                        out_h.at[cid, pl.ds(r0, _R_LEN)])

    return _agg


_agg128 = _make_agg(_D)


_BLK = 1000
_GRID = _N // _BLK


def _norm(deg2):
    return lax.rsqrt(jnp.maximum(deg2[0] + deg2[1], 1.0))


def _tc_a_body(x_ref, w_ref, do_ref, o_ref):
    ns = _norm(do_ref[...])
    m = jnp.dot(x_ref[...], w_ref[...], preferred_element_type=jnp.float32)
    o_ref[...] = m * ns


def _tc_bc_body(a_ref, di_ref, b_ref, w_ref, do_ref, o_ref):
    nd = _norm(di_ref[...])
    h = jnp.maximum((a_ref[0] + a_ref[1]) * nd + b_ref[...], 0.0)
    ns = _norm(do_ref[...])
    o_ref[...] = jnp.dot(h, w_ref[...], preferred_element_type=jnp.float32) * ns


def _tc_d_body(a_ref, di_ref, b_ref, o_ref):
    nd = _norm(di_ref[...])
    s = a_ref[0][:, :_NCLS] + a_ref[1][:, :_NCLS]
    o_ref[...] = s * nd + b_ref[...]


def _row_spec(d):
    return pl.BlockSpec((_BLK, d), lambda i: (i, 0))


def _pair_spec(d):
    return pl.BlockSpec((2, _BLK, d), lambda i: (0, i, 0))


def _full_spec(shape):
    return pl.BlockSpec(shape, lambda i: tuple(0 for _ in shape))


def _tc_a(x, w, dop):
    return pl.pallas_call(
        _tc_a_body,
        grid=(_GRID,),
        in_specs=[_row_spec(_D), _full_spec(w.shape), _pair_spec(1)],
        out_specs=_row_spec(_D),
        out_shape=jax.ShapeDtypeStruct((_N, _D), jnp.float32),
    )(x, w, dop)


def _tc_bc(a, dip, b2d, w, dop):
    d_out = w.shape[1]
    return pl.pallas_call(
        _tc_bc_body,
        grid=(_GRID,),
        in_specs=[_pair_spec(_D), _pair_spec(1), _full_spec(b2d.shape),
                  _full_spec(w.shape), _pair_spec(1)],
        out_specs=_row_spec(d_out),
        out_shape=jax.ShapeDtypeStruct((_N, d_out), jnp.float32),
    )(a, dip, b2d, w, dop)


def _tc_d(a, dip, b2d):
    return pl.pallas_call(
        _tc_d_body,
        grid=(_GRID,),
        in_specs=[_pair_spec(_D), _pair_spec(1), _full_spec(b2d.shape)],
        out_specs=_row_spec(_NCLS),
        out_shape=jax.ShapeDtypeStruct((_N, _NCLS), jnp.float32),
    )(a, dip, b2d)


def kernel(x, edge_index, W1, b1, W2, b2, W3, b3):
    src = edge_index[0]
    dst = edge_index[1]

    degs = _deg_kernel(src, dst).reshape(_NSC, 2, _N)
    dop = degs[:, 0, :, None]
    dip = degs[:, 1, :, None]

    zeros128 = jnp.zeros((_N, _D), jnp.float32)
    p1 = _tc_a(x, W1, dop)
    a1 = _agg128(p1, src, dst, zeros128)
    p2 = _tc_bc(a1, dip, b1.reshape(1, -1), W2, dop)
    a2 = _agg128(p2, src, dst, zeros128)
    w3p = jnp.concatenate([W3, jnp.zeros((_HID, _D - _NCLS), jnp.float32)], axis=1)
    p3 = _tc_bc(a2, dip, b2.reshape(1, -1), w3p, dop)
    a3 = _agg128(p3, src, dst, zeros128)
    return _tc_d(a3, dip, b3.reshape(1, -1))


# speedup vs baseline: 1.0066x; 1.0066x over previous
import functools

import jax
import jax.numpy as jnp
from jax import lax
from jax.experimental import pallas as pl
from jax.experimental.pallas import tpu as pltpu
from jax.experimental.pallas import tpu_sc as plsc

_N = 10000
_E = 320000
_D = 128
_HID = 128
_NCLS = 64

_NSC = 2
_NTEC = 16
_NW = _NSC * _NTEC
_EPT = _E // _NW
_C = 80
_ITERS = _EPT // _C
_NB = 3

_R_STEP = 624
_R_LEN = 640

_mesh = plsc.VectorSubcoreMesh(core_axis_name="c", subcore_axis_name="s")


@functools.partial(
    pl.kernel,
    out_type=jax.ShapeDtypeStruct((_NSC * 2 * _N,), jnp.float32),
    mesh=_mesh,
    scratch_types=[
        pltpu.VMEM((_EPT,), jnp.int32),
        pltpu.VMEM((_EPT,), jnp.int32),
        pltpu.VMEM((_C,), jnp.float32),
        pltpu.VMEM((_R_LEN,), jnp.float32),
        pltpu.VMEM_SHARED((_N,), jnp.float32),
        pltpu.VMEM_SHARED((_N,), jnp.float32),
        pltpu.SemaphoreType.DMA,
        pltpu.SemaphoreType.DMA,
    ],
)
def _deg_kernel(src_h, dst_h, out_h,
                sidx, didx, ones_v, stage_v, dego_sp, degi_sp, sem0, sem1):
    cid = lax.axis_index("c")
    sid = lax.axis_index("s")
    wid = sid * _NSC + cid
    r0 = pl.multiple_of(sid * _R_STEP, 8)
    ebase = pl.multiple_of(wid * _EPT, 8)
    pltpu.sync_copy(src_h.at[pl.ds(ebase, _EPT)], sidx)
    pltpu.sync_copy(dst_h.at[pl.ds(ebase, _EPT)], didx)
    for j in range(_C // 16):
        ones_v[pl.ds(j * 16, 16)] = jnp.ones((16,), jnp.float32)
    for j in range(_R_LEN // 16):
        stage_v[pl.ds(j * 16, 16)] = jnp.zeros((16,), jnp.float32)
    pltpu.sync_copy(stage_v, dego_sp.at[pl.ds(r0, _R_LEN)])
    pltpu.sync_copy(stage_v, degi_sp.at[pl.ds(r0, _R_LEN)])
    plsc.subcore_barrier()

    def chunk(i):
        return pl.ds(pl.multiple_of(i * _C, 8), _C)

    def body(i, carry):
        pltpu.async_copy(ones_v, dego_sp.at[sidx.at[chunk(i)]], sem0, add=True)
        pltpu.async_copy(ones_v, degi_sp.at[didx.at[chunk(i)]], sem1, add=True)
        return carry

    lax.fori_loop(0, _ITERS, body, 0)

    def drain(i, carry):
        pltpu.make_async_copy(ones_v, dego_sp.at[sidx.at[chunk(0)]], sem0).wait()
        pltpu.make_async_copy(ones_v, degi_sp.at[didx.at[chunk(0)]], sem1).wait()
        return carry

    lax.fori_loop(0, _ITERS, drain, 0)
    plsc.subcore_barrier()
    o_base = pl.multiple_of(cid * (2 * _N) + r0, 8)
    pltpu.sync_copy(dego_sp.at[pl.ds(r0, _R_LEN)], stage_v)
    pltpu.sync_copy(stage_v, out_h.at[pl.ds(o_base, _R_LEN)])
    pltpu.sync_copy(degi_sp.at[pl.ds(r0, _R_LEN)], stage_v)
    pltpu.sync_copy(stage_v, out_h.at[pl.ds(o_base + _N, _R_LEN)])


def _make_agg(d):
    @functools.partial(
        pl.kernel,
        out_type=jax.ShapeDtypeStruct((_NSC, _N, d), jnp.float32),
        mesh=_mesh,
        scratch_types=[
            pltpu.VMEM((_EPT,), jnp.int32),
            pltpu.VMEM((_EPT,), jnp.int32),
            [pltpu.VMEM((_C, d), jnp.float32) for _ in range(_NB)],
            [pltpu.SemaphoreType.DMA for _ in range(_NB)],
            [pltpu.SemaphoreType.DMA for _ in range(_NB)],
            pltpu.VMEM_SHARED((_N, d), jnp.float32),
        ],
    )
    def _agg(p_h, src_h, dst_h, out_h, sidx, didx, rows, gsem, ssem, agg_sp):
        cid = lax.axis_index("c")
        sid = lax.axis_index("s")
        wid = sid * _NSC + cid
        r0 = pl.multiple_of(sid * _R_STEP, 8)
        ebase = pl.multiple_of(wid * _EPT, 8)
        pltpu.sync_copy(src_h.at[pl.ds(ebase, _EPT)], sidx)
        pltpu.sync_copy(dst_h.at[pl.ds(ebase, _EPT)], didx)

        def zrow(r, carry):
            for j in range(d // 16):
                rows[0][r, pl.ds(j * 16, 16)] = jnp.zeros((16,), jnp.float32)
            return carry

        lax.fori_loop(0, _C, zrow, 0)
        for k in range(_R_LEN // _C):
            pltpu.sync_copy(rows[0], agg_sp.at[pl.ds(r0 + k * _C, _C)])
        plsc.subcore_barrier()

        def chunk(i):
            return pl.ds(pl.multiple_of(i * _C, 8), _C)

        def gstart(i, b):
            pltpu.async_copy(p_h.at[sidx.at[chunk(i)]], rows[b], gsem[b])

        def gwait(b):
            pltpu.make_async_copy(p_h.at[sidx.at[chunk(0)]], rows[b], gsem[b]).wait()

        def sstart(i, b):
            pltpu.async_copy(rows[b], agg_sp.at[didx.at[chunk(i)]], ssem[b],
                             add=True)

        def swait(b):
            pltpu.make_async_copy(rows[b], agg_sp.at[didx.at[chunk(0)]],
                                  ssem[b]).wait()

        gstart(0, 0)
        for i in range(2):
            gstart(i + 1, (i + 1) % _NB)
            gwait(i % _NB)
            sstart(i, i % _NB)

        def step(i, off):
            b = (2 + off) % _NB
            swait(off % _NB)
            gstart(i + 1, (3 + off) % _NB)
            gwait(b)
            sstart(i, b)

        def body(k, carry):
            for off in range(_NB):
                step(2 + _NB * k + off, off)
            return carry

        n_main = (_ITERS - 3 - 2) // _NB
        lax.fori_loop(0, n_main, body, 0)
        for j in range(_ITERS - 3 - _NB * n_main):
            step(2 + _NB * n_main + j, j)
        i_last = _ITERS - 1
        gwait(i_last % _NB)
        sstart(i_last, i_last % _NB)
        for b in range(_NB):
            swait(b)
        plsc.subcore_barrier()
        for k in range(_R_LEN // _C):
            pltpu.sync_copy(agg_sp.at[pl.ds(r0 + k * _C, _C)], rows[0])
            pltpu.sync_copy(rows[0], out_h.at[cid, pl.ds(r0 + k * _C, _C)])

    return _agg


_agg128 = _make_agg(_D)


_BLK = 1000
_GRID = _N // _BLK


def _norm(deg2):
    return lax.rsqrt(jnp.maximum(deg2[0] + deg2[1], 1.0))


def _tc_a_body(x_ref, w_ref, do_ref, o_ref):
    ns = _norm(do_ref[...])
    m = jnp.dot(x_ref[...], w_ref[...], preferred_element_type=jnp.float32)
    o_ref[...] = m * ns


def _tc_bc_body(a_ref, di_ref, b_ref, w_ref, do_ref, o_ref):
    nd = _norm(di_ref[...])
    h = jnp.maximum((a_ref[0] + a_ref[1]) * nd + b_ref[...], 0.0)
    ns = _norm(do_ref[...])
    o_ref[...] = jnp.dot(h, w_ref[...], preferred_element_type=jnp.float32) * ns


def _tc_d_body(a_ref, di_ref, b_ref, o_ref):
    nd = _norm(di_ref[...])
    s = a_ref[0][:, :_NCLS] + a_ref[1][:, :_NCLS]
    o_ref[...] = s * nd + b_ref[...]


def _row_spec(d):
    return pl.BlockSpec((_BLK, d), lambda i: (i, 0))


def _pair_spec(d):
    return pl.BlockSpec((2, _BLK, d), lambda i: (0, i, 0))


def _full_spec(shape):
    return pl.BlockSpec(shape, lambda i: tuple(0 for _ in shape))


def _tc_a(x, w, dop):
    return pl.pallas_call(
        _tc_a_body,
        grid=(_GRID,),
        in_specs=[_row_spec(_D), _full_spec(w.shape), _pair_spec(1)],
        out_specs=_row_spec(_D),
        out_shape=jax.ShapeDtypeStruct((_N, _D), jnp.float32),
    )(x, w, dop)


def _tc_bc(a, dip, b2d, w, dop):
    d_out = w.shape[1]
    return pl.pallas_call(
        _tc_bc_body,
        grid=(_GRID,),
        in_specs=[_pair_spec(_D), _pair_spec(1), _full_spec(b2d.shape),
                  _full_spec(w.shape), _pair_spec(1)],
        out_specs=_row_spec(d_out),
        out_shape=jax.ShapeDtypeStruct((_N, d_out), jnp.float32),
    )(a, dip, b2d, w, dop)


def _tc_d(a, dip, b2d):
    return pl.pallas_call(
        _tc_d_body,
        grid=(_GRID,),
        in_specs=[_pair_spec(_D), _pair_spec(1), _full_spec(b2d.shape)],
        out_specs=_row_spec(_NCLS),
        out_shape=jax.ShapeDtypeStruct((_N, _NCLS), jnp.float32),
    )(a, dip, b2d)


def kernel(x, edge_index, W1, b1, W2, b2, W3, b3):
    src = edge_index[0]
    dst = edge_index[1]

    degs = _deg_kernel(src, dst).reshape(_NSC, 2, _N)
    dop = degs[:, 0, :, None]
    dip = degs[:, 1, :, None]

    p1 = _tc_a(x, W1, dop)
    a1 = _agg128(p1, src, dst)
    p2 = _tc_bc(a1, dip, b1.reshape(1, -1), W2, dop)
    a2 = _agg128(p2, src, dst)
    w3p = jnp.concatenate([W3, jnp.zeros((_HID, _D - _NCLS), jnp.float32)], axis=1)
    p3 = _tc_bc(a2, dip, b2.reshape(1, -1), w3p, dop)
    a3 = _agg128(p3, src, dst)
    return _tc_d(a3, dip, b3.reshape(1, -1))
